# Initial kernel scaffold; baseline (speedup 1.0000x reference)
import jax
import jax.numpy as jnp
from jax.experimental import pallas as pl


def kernel(x, W):
    raise NotImplementedError("write your pallas kernel here")


# speedup vs baseline: 1.3320x; 1.3320x over previous
import functools

import jax
import jax.numpy as jnp
from jax import lax
from jax.experimental import pallas as pl
from jax.experimental.pallas import tpu as pltpu
from jax.experimental.pallas import tpu_sc as plsc

_DIM = 4096
_N_EXPERTS = 64
_TOPK = 8
_N_TOKENS = 32768
_BLOCK_T = 1024
_SC_CHUNK = 256
_LANES = 16


def _probs_block(x_ref, wt_ref, probs_ref):
    s = jnp.dot(x_ref[...], wt_ref[...], preferred_element_type=jnp.float32)
    m = jnp.max(s, axis=-1, keepdims=True)
    e = jnp.exp(s - m)
    denom = jnp.sum(e, axis=-1, keepdims=True)
    probs_ref[...] = e / denom


def _merge(ka, va, kb, vb):
    krb = lax.rev(kb, (0,))
    vrb = lax.rev(vb, (0,))
    take = ka >= krb
    km = jnp.where(take, ka, krb)
    vm = jnp.where(take, va, vrb)
    return plsc.sort_key_val(km, vm, descending=True)


def _make_sc_topk(n_tokens):
    info = plsc.get_sparse_core_info()
    n_workers = info.num_cores * info.num_subcores
    rows_per_w = n_tokens // n_workers
    n_chunks = rows_per_w // _SC_CHUNK
    mesh = plsc.VectorSubcoreMesh(core_axis_name="c", subcore_axis_name="s")

    @functools.partial(
        pl.kernel,
        mesh=mesh,
        out_type=[
            jax.ShapeDtypeStruct((n_tokens, _LANES), jnp.float32),
            jax.ShapeDtypeStruct((n_tokens, _LANES), jnp.int32),
        ],
        scratch_types=[
            pltpu.VMEM((_SC_CHUNK, _N_EXPERTS), jnp.float32),
            pltpu.VMEM((_SC_CHUNK, _LANES), jnp.float32),
            pltpu.VMEM((_SC_CHUNK, _LANES), jnp.int32),
        ],
        compiler_params=pltpu.CompilerParams(needs_layout_passes=False),
    )
    def sc_topk(probs_hbm, outv_hbm, outi_hbm, pv, ov, oi):
        wid = lax.axis_index("s") * info.num_cores + lax.axis_index("c")
        base = wid * rows_per_w
        iotas = [
            lax.broadcasted_iota(jnp.int32, (_LANES,), 0) + _LANES * k
            for k in range(_N_EXPERTS // _LANES)
        ]
        for c in range(n_chunks):
            off = base + c * _SC_CHUNK
            pltpu.sync_copy(probs_hbm.at[pl.ds(off, _SC_CHUNK)], pv)

            def body(t, carry):
                ks, vs = [], []
                for k in range(_N_EXPERTS // _LANES):
                    key = pv[t, pl.ds(_LANES * k, _LANES)]
                    sk, sv = plsc.sort_key_val(key, iotas[k], descending=True)
                    ks.append(sk)
                    vs.append(sv)
                k01, v01 = _merge(ks[0], vs[0], ks[1], vs[1])
                k23, v23 = _merge(ks[2], vs[2], ks[3], vs[3])
                kf, vf = _merge(k01, v01, k23, v23)
                ov[t, :] = kf
                oi[t, :] = vf
                return carry

            lax.fori_loop(0, _SC_CHUNK, body, 0)
            pltpu.sync_copy(ov, outv_hbm.at[pl.ds(off, _SC_CHUNK)])
            pltpu.sync_copy(oi, outi_hbm.at[pl.ds(off, _SC_CHUNK)])

    return sc_topk


@functools.partial(jax.jit, static_argnames=("interpret",))
def kernel(x, W, interpret=False):
    wt = W.T
    n_tokens = x.shape[0]
    block_t = min(_BLOCK_T, n_tokens)
    grid = (n_tokens // block_t,)
    probs = pl.pallas_call(
        _probs_block,
        grid=grid,
        in_specs=[
            pl.BlockSpec((block_t, _DIM), lambda i: (i, 0)),
            pl.BlockSpec((_DIM, _N_EXPERTS), lambda i: (0, 0)),
        ],
        out_specs=pl.BlockSpec((block_t, _N_EXPERTS), lambda i: (i, 0)),
        out_shape=jax.ShapeDtypeStruct((n_tokens, _N_EXPERTS), jnp.float32),
        interpret=interpret,
    )(x, wt)
    outv, outi = _make_sc_topk(n_tokens)(probs)
    return (outi[:, :_TOPK], outv[:, :_TOPK])


# speedup vs baseline: 1.5827x; 1.1882x over previous
import functools

import jax
import jax.numpy as jnp
from jax.experimental import pallas as pl
from jax.experimental.pallas import tpu as pltpu

_DIM = 4096
_N_EXPERTS = 64
_TOPK = 8
_N_TOKENS = 32768
_BLOCK_T = 1024


def _router_block(x_ref, wt_ref, topi_ref, topv_ref):
    s = jnp.dot(x_ref[...], wt_ref[...], preferred_element_type=jnp.float32)
    st = s.T
    m = jnp.max(st, axis=0, keepdims=True)
    e = jnp.exp(st - m)
    denom = jnp.sum(e, axis=0, keepdims=True)
    p = e / denom

    iota_f = jax.lax.broadcasted_iota(jnp.int32, p.shape, 0).astype(jnp.float32)
    work = p
    vals = []
    idxs = []
    for _ in range(_TOPK):
        mj = jnp.max(work, axis=0, keepdims=True)
        ij = jnp.min(
            jnp.where(work == mj, iota_f, float(_N_EXPERTS)),
            axis=0,
            keepdims=True,
        )
        vals.append(mj)
        idxs.append(ij)
        work = jnp.where(iota_f == ij, -1.0, work)
    topv_ref[...] = jnp.concatenate(vals, axis=0).T
    topi_ref[...] = jnp.concatenate(idxs, axis=0).T.astype(jnp.int32)


@functools.partial(jax.jit, static_argnames=("interpret",))
def kernel(x, W, interpret=False):
    wt = W.T
    n_tokens = x.shape[0]
    block_t = min(_BLOCK_T, n_tokens)
    grid = (n_tokens // block_t,)
    topi, topv = pl.pallas_call(
        _router_block,
        grid=grid,
        in_specs=[
            pl.BlockSpec((block_t, _DIM), lambda i: (i, 0)),
            pl.BlockSpec((_DIM, _N_EXPERTS), lambda i: (0, 0)),
        ],
        out_specs=[
            pl.BlockSpec((block_t, _TOPK), lambda i: (i, 0)),
            pl.BlockSpec((block_t, _TOPK), lambda i: (i, 0)),
        ],
        out_shape=[
            jax.ShapeDtypeStruct((n_tokens, _TOPK), jnp.int32),
            jax.ShapeDtypeStruct((n_tokens, _TOPK), jnp.float32),
        ],
        interpret=interpret,
    )(x, wt)
    return (topi, topv)
